# baseline (speedup 1.0000x reference)
import functools

import jax
import jax.numpy as jnp
from jax import lax
from jax.experimental import pallas as pl

NUM_LAYERS = 3
GRAPHS_PER_STEP = 2


def _dot(a, b):
    return lax.dot_general(a, b, (((1,), (0,)), ((), ())),
                           preferred_element_type=jnp.float32)


def _dot_t(a, b):
    return lax.dot_general(a, b, (((0,), (0,)), ((), ())),
                           preferred_element_type=jnp.float32)


def _ggnn_kernel(x_ref, adj_ref, wcat_ref, wih_ref, bih_ref, bhh_ref,
                 out_ref, *, num_layers, d, n, g):
    bf = jnp.bfloat16
    A = adj_ref[:, :, :].astype(jnp.float32).astype(bf)
    b_ih = bih_ref[:, :]
    b_hh = bhh_ref[:, :]
    wih = wih_ref[:, :]
    hs = [x_ref[i] for i in range(g)]
    for l in range(num_layers):
        hcat = jnp.concatenate([h.astype(bf) for h in hs], axis=0)
        cat = _dot(hcat, wcat_ref[l])
        for i in range(g):
            m = cat[i * n:(i + 1) * n, 0:d].astype(bf)
            gh = cat[i * n:(i + 1) * n, d:4 * d] + b_hh
            agg = _dot_t(A[i], m)
            gi = _dot(agg.astype(bf), wih) + b_ih
            r = 0.5 * jnp.tanh(0.5 * (gi[:, 0:d] + gh[:, 0:d])) + 0.5
            z = 0.5 * jnp.tanh(0.5 * (gi[:, d:2 * d] + gh[:, d:2 * d])) + 0.5
            nn = jnp.tanh(gi[:, 2 * d:3 * d] + r * gh[:, 2 * d:3 * d])
            hs[i] = hs[i] + (1.0 - z) * (nn - hs[i])
    for i in range(g):
        out_ref[i] = hs[i]


def kernel(x, adj, W, W_ih, W_hh, b_ih, b_hh):
    B, N, D = x.shape
    num_layers = W.shape[0]
    g = min(GRAPHS_PER_STEP, B)
    wcat = jnp.concatenate(
        [W, jnp.broadcast_to(W_hh.T[None], (num_layers, D, 3 * D))],
        axis=2).astype(jnp.bfloat16)
    wih_t = W_ih.T.astype(jnp.bfloat16)
    b_ih2 = b_ih.reshape(1, 3 * D)
    b_hh2 = b_hh.reshape(1, 3 * D)
    return pl.pallas_call(
        functools.partial(_ggnn_kernel, num_layers=num_layers, d=D, n=N, g=g),
        grid=(B // g,),
        in_specs=[
            pl.BlockSpec((g, N, D), lambda b: (b, 0, 0)),
            pl.BlockSpec((g, N, N), lambda b: (b, 0, 0)),
            pl.BlockSpec((num_layers, D, 4 * D), lambda b: (0, 0, 0)),
            pl.BlockSpec((D, 3 * D), lambda b: (0, 0)),
            pl.BlockSpec((1, 3 * D), lambda b: (0, 0)),
            pl.BlockSpec((1, 3 * D), lambda b: (0, 0)),
        ],
        out_specs=pl.BlockSpec((g, N, D), lambda b: (b, 0, 0)),
        out_shape=jax.ShapeDtypeStruct((B, N, D), jnp.float32),
    )(x, adj, wcat, wih_t, b_ih2, b_hh2)


# speedup vs baseline: 1.1261x; 1.1261x over previous
import functools

import jax
import jax.numpy as jnp
from jax import lax
from jax.experimental import pallas as pl

NUM_LAYERS = 3
GRAPHS_PER_STEP = 2


def _dot(a, b):
    return lax.dot_general(a, b, (((1,), (0,)), ((), ())),
                           preferred_element_type=jnp.float32)


def _dot_t(a, b):
    return lax.dot_general(a, b, (((0,), (0,)), ((), ())),
                           preferred_element_type=jnp.float32)


def _ggnn_kernel(x_ref, adj_ref, wcat_ref, wih_ref, bih_ref, bhh_ref,
                 out_ref, *, num_layers, d, n, g):
    bf = jnp.bfloat16
    h = x_ref[:, :, :].reshape(g * n, d)
    A = adj_ref[:, :, :].astype(jnp.float32).astype(bf)
    b_ih = bih_ref[:, :]
    b_hh = bhh_ref[:, :]
    wih = wih_ref[:, :]
    for l in range(num_layers):
        cat = _dot(h.astype(bf), wcat_ref[l])
        m = cat[:, 0:d].astype(bf)
        gh = cat[:, d:4 * d] + b_hh
        aggs = [_dot_t(A[i], m[i * n:(i + 1) * n, :]) for i in range(g)]
        agg = jnp.concatenate(aggs, axis=0)
        gi = _dot(agg.astype(bf), wih) + b_ih
        r = 0.5 * jnp.tanh(0.5 * (gi[:, 0:d] + gh[:, 0:d])) + 0.5
        z = 0.5 * jnp.tanh(0.5 * (gi[:, d:2 * d] + gh[:, d:2 * d])) + 0.5
        nn = jnp.tanh(gi[:, 2 * d:3 * d] + r * gh[:, 2 * d:3 * d])
        h = h + (1.0 - z) * (nn - h)
    out_ref[:, :, :] = h.reshape(g, n, d)


def kernel(x, adj, W, W_ih, W_hh, b_ih, b_hh):
    B, N, D = x.shape
    num_layers = W.shape[0]
    g = min(GRAPHS_PER_STEP, B)
    wcat = jnp.concatenate(
        [W, jnp.broadcast_to(W_hh.T[None], (num_layers, D, 3 * D))],
        axis=2).astype(jnp.bfloat16)
    wih_t = W_ih.T.astype(jnp.bfloat16)
    b_ih2 = b_ih.reshape(1, 3 * D)
    b_hh2 = b_hh.reshape(1, 3 * D)
    return pl.pallas_call(
        functools.partial(_ggnn_kernel, num_layers=num_layers, d=D, n=N, g=g),
        grid=(B // g,),
        in_specs=[
            pl.BlockSpec((g, N, D), lambda b: (b, 0, 0)),
            pl.BlockSpec((g, N, N), lambda b: (b, 0, 0)),
            pl.BlockSpec((num_layers, D, 4 * D), lambda b: (0, 0, 0)),
            pl.BlockSpec((D, 3 * D), lambda b: (0, 0)),
            pl.BlockSpec((1, 3 * D), lambda b: (0, 0)),
            pl.BlockSpec((1, 3 * D), lambda b: (0, 0)),
        ],
        out_specs=pl.BlockSpec((g, N, D), lambda b: (b, 0, 0)),
        out_shape=jax.ShapeDtypeStruct((B, N, D), jnp.float32),
    )(x, adj, wcat, wih_t, b_ih2, b_hh2)


# speedup vs baseline: 1.4341x; 1.2735x over previous
import functools

import jax
import jax.numpy as jnp
from jax import lax
from jax.experimental import pallas as pl

NUM_LAYERS = 3
GRAPHS_PER_STEP = 2


def _dot(a, b):
    return lax.dot_general(a, b, (((1,), (0,)), ((), ())),
                           preferred_element_type=jnp.float32)


def _dot_tl(a, b):
    return lax.dot_general(a, b, (((0,), (0,)), ((), ())),
                           preferred_element_type=jnp.float32)


def _dot_tr(a, b):
    return lax.dot_general(a, b, (((1,), (1,)), ((), ())),
                           preferred_element_type=jnp.float32)


def _ggnn_kernel(x_ref, adj_ref, w_ref, wih_ref, whh_ref, bih_ref, bhh_ref,
                 out_ref, *, num_layers, d, n, g):
    bf = jnp.bfloat16
    h = x_ref[:, :, :].reshape(g * n, d)
    A = adj_ref[:, :, :].astype(jnp.float32).astype(bf)
    b_ih = bih_ref[:, :]
    b_hh = bhh_ref[:, :]
    wih = wih_ref[:, :].astype(bf)
    whh = whh_ref[:, :].astype(bf)
    for l in range(num_layers):
        hb = h.astype(bf)
        m = _dot(hb, w_ref[l].astype(bf))
        gh = _dot_tr(hb, whh) + b_hh
        mb = m.astype(bf)
        aggs = [_dot_tl(A[i], mb[i * n:(i + 1) * n, :]) for i in range(g)]
        agg = jnp.concatenate(aggs, axis=0)
        gi = _dot_tr(agg.astype(bf), wih) + b_ih
        r = 0.5 * jnp.tanh(0.5 * (gi[:, 0:d] + gh[:, 0:d])) + 0.5
        z = 0.5 * jnp.tanh(0.5 * (gi[:, d:2 * d] + gh[:, d:2 * d])) + 0.5
        nn = jnp.tanh(gi[:, 2 * d:3 * d] + r * gh[:, 2 * d:3 * d])
        h = h + (1.0 - z) * (nn - h)
    out_ref[:, :, :] = h.reshape(g, n, d)


def kernel(x, adj, W, W_ih, W_hh, b_ih, b_hh):
    B, N, D = x.shape
    num_layers = W.shape[0]
    g = min(GRAPHS_PER_STEP, B)
    b_ih2 = b_ih.reshape(1, 3 * D)
    b_hh2 = b_hh.reshape(1, 3 * D)
    return pl.pallas_call(
        functools.partial(_ggnn_kernel, num_layers=num_layers, d=D, n=N, g=g),
        grid=(B // g,),
        in_specs=[
            pl.BlockSpec((g, N, D), lambda b: (b, 0, 0)),
            pl.BlockSpec((g, N, N), lambda b: (b, 0, 0)),
            pl.BlockSpec((num_layers, D, D), lambda b: (0, 0, 0)),
            pl.BlockSpec((3 * D, D), lambda b: (0, 0)),
            pl.BlockSpec((3 * D, D), lambda b: (0, 0)),
            pl.BlockSpec((1, 3 * D), lambda b: (0, 0)),
            pl.BlockSpec((1, 3 * D), lambda b: (0, 0)),
        ],
        out_specs=pl.BlockSpec((g, N, D), lambda b: (b, 0, 0)),
        out_shape=jax.ShapeDtypeStruct((B, N, D), jnp.float32),
    )(x, adj, W, W_ih, W_hh, b_ih2, b_hh2)
